# baseline (speedup 1.0000x reference)
import jax
import jax.numpy as jnp
from jax import lax
from jax.experimental import pallas as pl
from jax.experimental.pallas import tpu as pltpu
from jax.experimental.pallas import tpu_sc as plsc

_NC = 2
_NS = 16
_NW = _NC * _NS
_L = 16

_B, _S, _D = 128, 32, 2048
_CB = 16
_NCHUNK = _B // _CB
_NBUF = 3
_LEAD = 2


def _sc_body(x_hbm, t_hbm, o_hbm, trow, bufs, in_sems, out_sems):
    wid = lax.axis_index("s") * _NC + lax.axis_index("c")

    pltpu.sync_copy(t_hbm.at[wid], trow)

    def in_copy(c, j):
        return pltpu.make_async_copy(
            x_hbm.at[pl.ds(c * _CB, _CB), wid], bufs.at[j], in_sems.at[j])

    def out_copy(c, j):
        return pltpu.make_async_copy(
            bufs.at[j], o_hbm.at[pl.ds(c * _CB, _CB), wid], out_sems.at[j])

    for p in range(_LEAD):
        in_copy(p, p).start()

    def body(c, carry):
        j = lax.rem(c, _NBUF)
        in_copy(c, j).wait()

        @plsc.parallel_loop(0, _D, _L)
        def _(col):
            tv = trow[pl.ds(col, _L)]
            for r in range(_CB):
                plsc.addupdate(bufs.at[j, r, pl.ds(col, _L)], tv)

        out_copy(c, j).start()

        nxt = c + _LEAD
        jn = lax.rem(nxt, _NBUF)

        @pl.when(nxt < _NCHUNK)
        def _():
            @pl.when(nxt >= _NBUF)
            def _():
                out_copy(nxt - _NBUF, jn).wait()

            in_copy(nxt, jn).start()

        return carry

    lax.fori_loop(0, _NCHUNK, body, 0)

    for c in range(_NCHUNK - _NBUF, _NCHUNK):
        out_copy(c, c % _NBUF).wait()


@jax.jit
def _sc_add(x, pos_table):
    mesh = plsc.VectorSubcoreMesh(core_axis_name="c", subcore_axis_name="s")
    body = lambda x_hbm, t_hbm, o_hbm, trow, bufs, in_sems, out_sems: (
        _sc_body(x_hbm, t_hbm, o_hbm, trow, bufs, in_sems, out_sems))
    return pl.kernel(
        body,
        out_type=jax.ShapeDtypeStruct((_B, _S, _D), jnp.float32),
        mesh=mesh,
        scratch_types=[
            pltpu.VMEM((_D,), jnp.float32),
            pltpu.VMEM((_NBUF, _CB, _D), jnp.float32),
            pltpu.SemaphoreType.DMA((_NBUF,)),
            pltpu.SemaphoreType.DMA((_NBUF,)),
        ],
    )(x, pos_table)


def kernel(x, pos_table):
    return _sc_add(x, pos_table)


# speedup vs baseline: 1.0583x; 1.0583x over previous
import jax
import jax.numpy as jnp
from jax import lax
from jax.experimental import pallas as pl
from jax.experimental.pallas import tpu as pltpu
from jax.experimental.pallas import tpu_sc as plsc

_NC = 2
_NS = 16
_NW = _NC * _NS
_L = 16

_B, _S, _D = 128, 32, 2048
_CB = 8
_NCHUNK = _B // _CB
_NBUF = 7
_LEAD = 6


def _sc_body(x_hbm, t_hbm, o_hbm, trow, bufs, in_sems, out_sems):
    wid = lax.axis_index("s") * _NC + lax.axis_index("c")

    pltpu.sync_copy(t_hbm.at[wid], trow)

    def in_copy(c, j):
        return pltpu.make_async_copy(
            x_hbm.at[pl.ds(c * _CB, _CB), wid], bufs.at[j], in_sems.at[j])

    def out_copy(c, j):
        return pltpu.make_async_copy(
            bufs.at[j], o_hbm.at[pl.ds(c * _CB, _CB), wid], out_sems.at[j])

    for p in range(_LEAD):
        in_copy(p, p).start()

    def body(c, carry):
        j = lax.rem(c, _NBUF)
        in_copy(c, j).wait()

        @plsc.parallel_loop(0, _D, _L)
        def _(col):
            tv = trow[pl.ds(col, _L)]
            for r in range(_CB):
                plsc.addupdate(bufs.at[j, r, pl.ds(col, _L)], tv)

        out_copy(c, j).start()

        nxt = c + _LEAD
        jn = lax.rem(nxt, _NBUF)

        @pl.when(nxt < _NCHUNK)
        def _():
            @pl.when(nxt >= _NBUF)
            def _():
                out_copy(nxt - _NBUF, jn).wait()

            in_copy(nxt, jn).start()

        return carry

    lax.fori_loop(0, _NCHUNK, body, 0)

    for c in range(_NCHUNK - _NBUF, _NCHUNK):
        out_copy(c, c % _NBUF).wait()


@jax.jit
def _sc_add(x, pos_table):
    mesh = plsc.VectorSubcoreMesh(core_axis_name="c", subcore_axis_name="s")
    body = lambda x_hbm, t_hbm, o_hbm, trow, bufs, in_sems, out_sems: (
        _sc_body(x_hbm, t_hbm, o_hbm, trow, bufs, in_sems, out_sems))
    return pl.kernel(
        body,
        out_type=jax.ShapeDtypeStruct((_B, _S, _D), jnp.float32),
        mesh=mesh,
        scratch_types=[
            pltpu.VMEM((_D,), jnp.float32),
            pltpu.VMEM((_NBUF, _CB, _D), jnp.float32),
            pltpu.SemaphoreType.DMA((_NBUF,)),
            pltpu.SemaphoreType.DMA((_NBUF,)),
        ],
    )(x, pos_table)


def kernel(x, pos_table):
    return _sc_add(x, pos_table)
